# baseline (speedup 1.0000x reference)
import functools

import jax
import jax.numpy as jnp
from jax import lax
from jax.experimental import pallas as pl
from jax.experimental.pallas import tpu as pltpu
from jax.experimental.pallas import tpu_sc as plsc

NC = 2
NS = 16
NW = NC * NS
L = 16


def _sc_scatter_gather(n_nodes, n_edges, d, c_sz, sc_chunks, nbuf):
    epw = n_edges // NW
    n_chunks = epw // c_sz
    n_super = n_chunks // sc_chunks
    s_sz = sc_chunks * c_sz
    rows_pt = (n_nodes // NS) // 8 * 8
    tail = n_nodes - rows_pt * NS
    zrows = 8
    mesh = plsc.VectorSubcoreMesh(core_axis_name="c", subcore_axis_name="s")

    @functools.partial(
        pl.kernel,
        out_type=jax.ShapeDtypeStruct((NC * n_nodes, d), jnp.float32),
        mesh=mesh,
        scratch_types=(
            [
                pltpu.VMEM((sc_chunks, c_sz), jnp.int32),
                pltpu.VMEM((sc_chunks, c_sz), jnp.int32),
                pltpu.VMEM((s_sz,), jnp.float32),
                pltpu.VMEM((s_sz,), jnp.float32),
            ]
            + [pltpu.VMEM((c_sz, d), jnp.float32)] * nbuf
            + [
                pltpu.VMEM((zrows, d), jnp.float32),
                pltpu.VMEM_SHARED((n_nodes, d), jnp.float32),
            ]
            + [pltpu.SemaphoreType.DMA] * nbuf
            + [pltpu.SemaphoreType.DMA]
            + [pltpu.SemaphoreType.DMA]
        ),
    )
    def k(inp_hbm, sidx_hbm, tidx_hbm, enorm_hbm, esgn_hbm, out_hbm, *scr):
        sidx_v, tidx_v, w_v, sg_v = scr[:4]
        rows = scr[4:4 + nbuf]
        zbuf = scr[4 + nbuf]
        acc = scr[5 + nbuf]
        gs = scr[6 + nbuf:6 + 2 * nbuf]
        aux_sem = scr[6 + 2 * nbuf]
        stg_sem = scr[7 + 2 * nbuf]
        cid = lax.axis_index("c")
        sid = lax.axis_index("s")
        wid = sid * NC + cid
        row0 = pl.multiple_of(sid * rows_pt, 8)
        scope = jax.named_scope

        n_zcopies = rows_pt // zrows
        def zero_zbuf(r, _):
            for f in range(d // L):
                zbuf[r, pl.ds(f * L, L)] = jnp.zeros((L,), jnp.float32)
            return 0
        lax.fori_loop(0, zrows, zero_zbuf, 0)

        def zero_acc(i, _):
            pltpu.async_copy(
                zbuf, acc.at[pl.ds(pl.multiple_of(row0 + i * zrows, 8), zrows)],
                aux_sem)
            return 0
        lax.fori_loop(0, n_zcopies, zero_acc, 0)
        if tail:
            @pl.when(sid == NS - 1)
            def _():
                for tpart in range(0, tail, zrows):
                    pltpu.async_copy(
                        zbuf, acc.at[pl.ds(NS * rows_pt + tpart, zrows)],
                        aux_sem)

        def drain_zero(i, _):
            pltpu.make_async_copy(
                zbuf, acc.at[pl.ds(0, zrows)], aux_sem).wait()
            return 0

        def scale(rows_v, c):
            @plsc.parallel_loop(0, c_sz // L)
            def escale(g):
                w16 = w_v[pl.ds(c * c_sz + g * L, L)]
                for j in range(L):
                    wb = lax.gather(
                        w16, jnp.full((L, 1), j, jnp.int32),
                        lax.GatherDimensionNumbers(
                            offset_dims=(), collapsed_slice_dims=(0,),
                            start_index_map=(0,)),
                        (1,), mode=lax.GatherScatterMode.PROMISE_IN_BOUNDS)
                    e = g * L + j
                    for f in range(d // L):
                        rows_v[e, pl.ds(f * L, L)] = (
                            rows_v[e, pl.ds(f * L, L)] * wb)

        def issue_gather(c, rows_v, sem):
            pltpu.async_copy(inp_hbm.at[sidx_v.at[c]], rows_v, sem)

        def wait_gather(rows_v, sem):
            pltpu.make_async_copy(inp_hbm.at[sidx_v.at[0]], rows_v, sem).wait()

        def issue_scatter(c, rows_v, sem):
            pltpu.sync_copy(rows_v, acc.at[tidx_v.at[c]], add=True)

        def super_round(s, _):
            e0 = pl.multiple_of(wid * epw + s * s_sz, 8)
            pltpu.async_copy(sidx_hbm.at[wid, s], sidx_v, stg_sem)
            pltpu.async_copy(tidx_hbm.at[wid, s], tidx_v, stg_sem)
            pltpu.async_copy(enorm_hbm.at[pl.ds(e0, s_sz)], w_v, stg_sem)
            pltpu.async_copy(esgn_hbm.at[pl.ds(e0, s_sz)], sg_v, stg_sem)
            pltpu.make_async_copy(sidx_hbm.at[wid, 0], sidx_v, stg_sem).wait()
            pltpu.make_async_copy(tidx_hbm.at[wid, 0], tidx_v, stg_sem).wait()
            pltpu.make_async_copy(
                enorm_hbm.at[pl.ds(0, s_sz)], w_v, stg_sem).wait()
            pltpu.make_async_copy(
                esgn_hbm.at[pl.ds(0, s_sz)], sg_v, stg_sem).wait()

            def wmul(kk, _):
                w_v[pl.ds(kk * L, L)] = (
                    w_v[pl.ds(kk * L, L)] * sg_v[pl.ds(kk * L, L)])
                return 0
            lax.fori_loop(0, s_sz // L, wmul, 0)

            for b in range(min(nbuf - 1, sc_chunks)):
                issue_gather(b, rows[b], gs[b])

            @pl.when(s == 0)
            def _():
                lax.fori_loop(0, n_zcopies, drain_zero, 0)
                if tail:
                    @pl.when(sid == NS - 1)
                    def _():
                        for _i in range(0, tail, zrows):
                            pltpu.make_async_copy(
                                zbuf, acc.at[pl.ds(0, zrows)], aux_sem).wait()
                plsc.subcore_barrier()

            def phase(c, b):
                @pl.when(c < sc_chunks)
                def _():
                    wait_gather(rows[b], gs[b])
                    scale(rows[b], c)
                    b2 = (b + nbuf - 1) % nbuf
                    @pl.when(c + nbuf - 1 < sc_chunks)
                    def _():
                        issue_gather(c + nbuf - 1, rows[b2], gs[b2])
                    issue_scatter(c, rows[b], None)

            def group(t, _):
                for b in range(nbuf):
                    phase(nbuf * t + b, b)
                return 0
            lax.fori_loop(0, (sc_chunks + nbuf - 1) // nbuf, group, 0)
            return 0
        lax.fori_loop(0, n_super, super_round, 0)

        plsc.subcore_barrier()

        with scope("sc_writeout"):
            pltpu.sync_copy(
                acc.at[pl.ds(row0, rows_pt)],
                out_hbm.at[pl.ds(
                    pl.multiple_of(cid * n_nodes + row0, 8), rows_pt)])
            if tail:
                @pl.when(sid == NS - 1)
                def _():
                    pltpu.sync_copy(
                        acc.at[pl.ds(NS * rows_pt, tail)],
                        out_hbm.at[pl.ds(pl.multiple_of(
                            cid * n_nodes + NS * rows_pt, 8), tail)])

    return k


def _tc_add(n_nodes, d, blk):
    def body(a_ref, b_ref, o_ref):
        o_ref[...] = a_ref[...] + b_ref[...]

    return pl.pallas_call(
        body,
        grid=(n_nodes // blk,),
        in_specs=[pl.BlockSpec((blk, d), lambda i: (i, 0))] * 2,
        out_specs=pl.BlockSpec((blk, d), lambda i: (i, 0)),
        out_shape=jax.ShapeDtypeStruct((n_nodes, d), jnp.float32),
    )


@jax.jit
def kernel(input, sidx, tidx, enorm, esgn):
    n_nodes, d = input.shape
    n_edges = sidx.shape[0]
    c_sz = 80
    sc_chunks = 25

    n_super = n_edges // NW // c_sz // sc_chunks
    sidx3 = sidx.astype(jnp.int32).reshape(NW, n_super, sc_chunks, c_sz)
    tidx3 = tidx.astype(jnp.int32).reshape(NW, n_super, sc_chunks, c_sz)

    partials = _sc_scatter_gather(n_nodes, n_edges, d, c_sz, sc_chunks, 3)(
        input, sidx3, tidx3, enorm, esgn)
    return _tc_add(n_nodes, d, 1000)(partials[:n_nodes], partials[n_nodes:])


# speedup vs baseline: 1.1762x; 1.1762x over previous
import functools

import jax
import jax.numpy as jnp
from jax import lax
from jax.experimental import pallas as pl
from jax.experimental.pallas import tpu as pltpu
from jax.experimental.pallas import tpu_sc as plsc

NC = 2
NS = 16
NW = NC * NS
L = 16


def _sc_scatter_gather(n_nodes, n_edges, d, c_sz, sc_chunks, nbuf):
    epw = n_edges // NW
    n_chunks = epw // c_sz
    n_super = n_chunks // sc_chunks
    s_sz = sc_chunks * c_sz
    rows_pt = (n_nodes // NS) // 8 * 8
    tail = n_nodes - rows_pt * NS
    zrows = 8
    mesh = plsc.VectorSubcoreMesh(core_axis_name="c", subcore_axis_name="s")

    @functools.partial(
        pl.kernel,
        out_type=jax.ShapeDtypeStruct((NC * n_nodes, d), jnp.float32),
        mesh=mesh,
        scratch_types=(
            [
                pltpu.VMEM((sc_chunks, c_sz), jnp.int32),
                pltpu.VMEM((sc_chunks, c_sz), jnp.int32),
                pltpu.VMEM((s_sz,), jnp.float32),
                pltpu.VMEM((s_sz,), jnp.float32),
            ]
            + [pltpu.VMEM((c_sz, d), jnp.float32)] * nbuf
            + [
                pltpu.VMEM((zrows, d), jnp.float32),
                pltpu.VMEM_SHARED((n_nodes, d), jnp.float32),
            ]
            + [pltpu.SemaphoreType.DMA] * nbuf
            + [pltpu.SemaphoreType.DMA]
            + [pltpu.SemaphoreType.DMA]
        ),
    )
    def k(inp_hbm, sidx_hbm, tidx_hbm, enorm_hbm, esgn_hbm, out_hbm, *scr):
        sidx_v, tidx_v, w_v, sg_v = scr[:4]
        rows = scr[4:4 + nbuf]
        zbuf = scr[4 + nbuf]
        acc = scr[5 + nbuf]
        gs = scr[6 + nbuf:6 + 2 * nbuf]
        aux_sem = scr[6 + 2 * nbuf]
        stg_sem = scr[7 + 2 * nbuf]
        cid = lax.axis_index("c")
        sid = lax.axis_index("s")
        wid = sid * NC + cid
        row0 = pl.multiple_of(sid * rows_pt, 8)
        scope = jax.named_scope

        n_zcopies = rows_pt // zrows
        def zero_zbuf(r, _):
            for f in range(d // L):
                zbuf[r, pl.ds(f * L, L)] = jnp.zeros((L,), jnp.float32)
            return 0
        lax.fori_loop(0, zrows, zero_zbuf, 0)

        def zero_acc(i, _):
            pltpu.async_copy(
                zbuf, acc.at[pl.ds(pl.multiple_of(row0 + i * zrows, 8), zrows)],
                aux_sem)
            return 0
        lax.fori_loop(0, n_zcopies, zero_acc, 0)
        if tail:
            @pl.when(sid == NS - 1)
            def _():
                for tpart in range(0, tail, zrows):
                    pltpu.async_copy(
                        zbuf, acc.at[pl.ds(NS * rows_pt + tpart, zrows)],
                        aux_sem)

        def drain_zero(i, _):
            pltpu.make_async_copy(
                zbuf, acc.at[pl.ds(0, zrows)], aux_sem).wait()
            return 0

        def scale(rows_v, c):
            def escale(g, _):
                w16 = w_v[pl.ds(c * c_sz + g * L, L)]
                for j in range(L):
                    wb = lax.gather(
                        w16, jnp.full((L, 1), j, jnp.int32),
                        lax.GatherDimensionNumbers(
                            offset_dims=(), collapsed_slice_dims=(0,),
                            start_index_map=(0,)),
                        (1,), mode=lax.GatherScatterMode.PROMISE_IN_BOUNDS)
                    e = g * L + j
                    for f in range(d // L):
                        rows_v[e, pl.ds(f * L, L)] = (
                            rows_v[e, pl.ds(f * L, L)] * wb)
                return 0
            lax.fori_loop(0, c_sz // L, escale, 0)

        def issue_gather(c, rows_v, sem):
            pltpu.async_copy(inp_hbm.at[sidx_v.at[c]], rows_v, sem)

        def wait_gather(rows_v, sem):
            pltpu.make_async_copy(inp_hbm.at[sidx_v.at[0]], rows_v, sem).wait()

        def issue_scatter(c, rows_v, sem):
            pltpu.sync_copy(rows_v, acc.at[tidx_v.at[c]], add=True)

        def super_round(s, _):
            e0 = pl.multiple_of(wid * epw + s * s_sz, 8)
            pltpu.async_copy(sidx_hbm.at[wid, s], sidx_v, stg_sem)
            pltpu.async_copy(tidx_hbm.at[wid, s], tidx_v, stg_sem)
            pltpu.async_copy(enorm_hbm.at[pl.ds(e0, s_sz)], w_v, stg_sem)
            pltpu.async_copy(esgn_hbm.at[pl.ds(e0, s_sz)], sg_v, stg_sem)
            pltpu.make_async_copy(sidx_hbm.at[wid, 0], sidx_v, stg_sem).wait()
            pltpu.make_async_copy(tidx_hbm.at[wid, 0], tidx_v, stg_sem).wait()
            pltpu.make_async_copy(
                enorm_hbm.at[pl.ds(0, s_sz)], w_v, stg_sem).wait()
            pltpu.make_async_copy(
                esgn_hbm.at[pl.ds(0, s_sz)], sg_v, stg_sem).wait()

            def wmul(kk, _):
                w_v[pl.ds(kk * L, L)] = (
                    w_v[pl.ds(kk * L, L)] * sg_v[pl.ds(kk * L, L)])
                return 0
            lax.fori_loop(0, s_sz // L, wmul, 0)

            for b in range(min(nbuf - 1, sc_chunks)):
                issue_gather(b, rows[b], gs[b])

            @pl.when(s == 0)
            def _():
                lax.fori_loop(0, n_zcopies, drain_zero, 0)
                if tail:
                    @pl.when(sid == NS - 1)
                    def _():
                        for _i in range(0, tail, zrows):
                            pltpu.make_async_copy(
                                zbuf, acc.at[pl.ds(0, zrows)], aux_sem).wait()
                plsc.subcore_barrier()

            def phase(c, b):
                @pl.when(c < sc_chunks)
                def _():
                    b2 = (b + nbuf - 1) % nbuf
                    @pl.when(c + nbuf - 1 < sc_chunks)
                    def _():
                        issue_gather(c + nbuf - 1, rows[b2], gs[b2])
                    wait_gather(rows[b], gs[b])
                    scale(rows[b], c)
                    issue_scatter(c, rows[b], None)

            def group(t, _):
                for b in range(nbuf):
                    phase(nbuf * t + b, b)
                return 0
            lax.fori_loop(0, (sc_chunks + nbuf - 1) // nbuf, group, 0)
            return 0
        lax.fori_loop(0, n_super, super_round, 0)

        plsc.subcore_barrier()

        with scope("sc_writeout"):
            pltpu.sync_copy(
                acc.at[pl.ds(row0, rows_pt)],
                out_hbm.at[pl.ds(
                    pl.multiple_of(cid * n_nodes + row0, 8), rows_pt)])
            if tail:
                @pl.when(sid == NS - 1)
                def _():
                    pltpu.sync_copy(
                        acc.at[pl.ds(NS * rows_pt, tail)],
                        out_hbm.at[pl.ds(pl.multiple_of(
                            cid * n_nodes + NS * rows_pt, 8), tail)])

    return k


def _tc_add(n_nodes, d, blk):
    def body(a_ref, b_ref, o_ref):
        o_ref[...] = a_ref[...] + b_ref[...]

    return pl.pallas_call(
        body,
        grid=(n_nodes // blk,),
        in_specs=[pl.BlockSpec((blk, d), lambda i: (i, 0))] * 2,
        out_specs=pl.BlockSpec((blk, d), lambda i: (i, 0)),
        out_shape=jax.ShapeDtypeStruct((n_nodes, d), jnp.float32),
    )


@jax.jit
def kernel(input, sidx, tidx, enorm, esgn):
    n_nodes, d = input.shape
    n_edges = sidx.shape[0]
    c_sz = 80
    sc_chunks = 25

    n_super = n_edges // NW // c_sz // sc_chunks
    sidx3 = sidx.astype(jnp.int32).reshape(NW, n_super, sc_chunks, c_sz)
    tidx3 = tidx.astype(jnp.int32).reshape(NW, n_super, sc_chunks, c_sz)

    partials = _sc_scatter_gather(n_nodes, n_edges, d, c_sz, sc_chunks, 3)(
        input, sidx3, tidx3, enorm, esgn)
    return _tc_add(n_nodes, d, 2000)(partials[:n_nodes], partials[n_nodes:])


# speedup vs baseline: 1.1763x; 1.0001x over previous
import functools

import jax
import jax.numpy as jnp
from jax import lax
from jax.experimental import pallas as pl
from jax.experimental.pallas import tpu as pltpu
from jax.experimental.pallas import tpu_sc as plsc

NC = 2
NS = 16
NW = NC * NS
L = 16


def _sc_scatter_gather(n_nodes, n_edges, d, c_sz, sc_chunks, nbuf):
    epw = n_edges // NW
    n_chunks = epw // c_sz
    n_super = n_chunks // sc_chunks
    s_sz = sc_chunks * c_sz
    rows_pt = (n_nodes // NS) // 8 * 8
    tail = n_nodes - rows_pt * NS
    zrows = 8
    mesh = plsc.VectorSubcoreMesh(core_axis_name="c", subcore_axis_name="s")

    @functools.partial(
        pl.kernel,
        out_type=jax.ShapeDtypeStruct((NC * n_nodes, d), jnp.float32),
        mesh=mesh,
        scratch_types=(
            [
                pltpu.VMEM((sc_chunks, c_sz), jnp.int32),
                pltpu.VMEM((sc_chunks, c_sz), jnp.int32),
                pltpu.VMEM((s_sz,), jnp.float32),
                pltpu.VMEM((s_sz,), jnp.float32),
            ]
            + [pltpu.VMEM((c_sz, d), jnp.float32)] * nbuf
            + [
                pltpu.VMEM((zrows, d), jnp.float32),
                pltpu.VMEM_SHARED((n_nodes, d), jnp.float32),
            ]
            + [pltpu.SemaphoreType.DMA] * nbuf
            + [pltpu.SemaphoreType.DMA]
            + [pltpu.SemaphoreType.DMA]
        ),
    )
    def k(inp_hbm, sidx_hbm, tidx_hbm, enorm_hbm, esgn_hbm, out_hbm, *scr):
        sidx_v, tidx_v, w_v, sg_v = scr[:4]
        rows = scr[4:4 + nbuf]
        zbuf = scr[4 + nbuf]
        acc = scr[5 + nbuf]
        gs = scr[6 + nbuf:6 + 2 * nbuf]
        aux_sem = scr[6 + 2 * nbuf]
        stg_sem = scr[7 + 2 * nbuf]
        cid = lax.axis_index("c")
        sid = lax.axis_index("s")
        wid = sid * NC + cid
        row0 = pl.multiple_of(sid * rows_pt, 8)

        n_zcopies = rows_pt // zrows
        def zero_zbuf(r, _):
            for f in range(d // L):
                zbuf[r, pl.ds(f * L, L)] = jnp.zeros((L,), jnp.float32)
            return 0
        lax.fori_loop(0, zrows, zero_zbuf, 0)

        def zero_acc(i, _):
            pltpu.async_copy(
                zbuf, acc.at[pl.ds(pl.multiple_of(row0 + i * zrows, 8), zrows)],
                aux_sem)
            return 0
        lax.fori_loop(0, n_zcopies, zero_acc, 0)
        if tail:
            @pl.when(sid == NS - 1)
            def _():
                for tpart in range(0, tail, zrows):
                    pltpu.async_copy(
                        zbuf, acc.at[pl.ds(NS * rows_pt + tpart, zrows)],
                        aux_sem)

        def drain_zero(i, _):
            pltpu.make_async_copy(
                zbuf, acc.at[pl.ds(0, zrows)], aux_sem).wait()
            return 0

        def scale(rows_v, c):
            def escale(g, _):
                w16 = w_v[pl.ds(c * c_sz + g * L, L)]
                for j in range(L):
                    wb = lax.gather(
                        w16, jnp.full((L, 1), j, jnp.int32),
                        lax.GatherDimensionNumbers(
                            offset_dims=(), collapsed_slice_dims=(0,),
                            start_index_map=(0,)),
                        (1,), mode=lax.GatherScatterMode.PROMISE_IN_BOUNDS)
                    e = g * L + j
                    for f in range(d // L):
                        rows_v[e, pl.ds(f * L, L)] = (
                            rows_v[e, pl.ds(f * L, L)] * wb)
                return 0
            lax.fori_loop(0, c_sz // L, escale, 0)

        def issue_gather(c, rows_v, sem):
            pltpu.async_copy(inp_hbm.at[sidx_v.at[c]], rows_v, sem)

        def wait_gather(rows_v, sem):
            pltpu.make_async_copy(inp_hbm.at[sidx_v.at[0]], rows_v, sem).wait()

        def issue_scatter(c, rows_v, sem):
            pltpu.sync_copy(rows_v, acc.at[tidx_v.at[c]], add=True)

        def super_round(s, _):
            e0 = pl.multiple_of(wid * epw + s * s_sz, 8)
            pltpu.async_copy(sidx_hbm.at[wid, s], sidx_v, stg_sem)
            pltpu.async_copy(tidx_hbm.at[wid, s], tidx_v, stg_sem)
            pltpu.async_copy(enorm_hbm.at[pl.ds(e0, s_sz)], w_v, stg_sem)
            pltpu.async_copy(esgn_hbm.at[pl.ds(e0, s_sz)], sg_v, stg_sem)
            pltpu.make_async_copy(sidx_hbm.at[wid, 0], sidx_v, stg_sem).wait()
            pltpu.make_async_copy(tidx_hbm.at[wid, 0], tidx_v, stg_sem).wait()
            pltpu.make_async_copy(
                enorm_hbm.at[pl.ds(0, s_sz)], w_v, stg_sem).wait()
            pltpu.make_async_copy(
                esgn_hbm.at[pl.ds(0, s_sz)], sg_v, stg_sem).wait()

            def wmul(kk, _):
                w_v[pl.ds(kk * L, L)] = (
                    w_v[pl.ds(kk * L, L)] * sg_v[pl.ds(kk * L, L)])
                return 0
            lax.fori_loop(0, s_sz // L, wmul, 0)

            for b in range(min(nbuf - 1, sc_chunks)):
                issue_gather(b, rows[b], gs[b])

            @pl.when(s == 0)
            def _():
                lax.fori_loop(0, n_zcopies, drain_zero, 0)
                if tail:
                    @pl.when(sid == NS - 1)
                    def _():
                        for _i in range(0, tail, zrows):
                            pltpu.make_async_copy(
                                zbuf, acc.at[pl.ds(0, zrows)], aux_sem).wait()
                plsc.subcore_barrier()

            def phase(c, b):
                @pl.when(c < sc_chunks)
                def _():
                    b2 = (b + nbuf - 1) % nbuf
                    @pl.when(c + nbuf - 1 < sc_chunks)
                    def _():
                        issue_gather(c + nbuf - 1, rows[b2], gs[b2])
                    wait_gather(rows[b], gs[b])
                    scale(rows[b], c)
                    issue_scatter(c, rows[b], None)

            def group(t, _):
                for b in range(nbuf):
                    phase(nbuf * t + b, b)
                return 0
            lax.fori_loop(0, (sc_chunks + nbuf - 1) // nbuf, group, 0)
            return 0
        lax.fori_loop(0, n_super, super_round, 0)

        plsc.subcore_barrier()

        pltpu.sync_copy(
            acc.at[pl.ds(row0, rows_pt)],
            out_hbm.at[pl.ds(
                pl.multiple_of(cid * n_nodes + row0, 8), rows_pt)])
        if tail:
            @pl.when(sid == NS - 1)
            def _():
                pltpu.sync_copy(
                    acc.at[pl.ds(NS * rows_pt, tail)],
                    out_hbm.at[pl.ds(pl.multiple_of(
                        cid * n_nodes + NS * rows_pt, 8), tail)])

    return k


def _tc_add(n_nodes, d, blk):
    def body(a_ref, b_ref, o_ref):
        o_ref[...] = a_ref[...] + b_ref[...]

    return pl.pallas_call(
        body,
        grid=(n_nodes // blk,),
        in_specs=[pl.BlockSpec((blk, d), lambda i: (i, 0))] * 2,
        out_specs=pl.BlockSpec((blk, d), lambda i: (i, 0)),
        out_shape=jax.ShapeDtypeStruct((n_nodes, d), jnp.float32),
    )


@jax.jit
def kernel(input, sidx, tidx, enorm, esgn):
    n_nodes, d = input.shape
    n_edges = sidx.shape[0]
    c_sz = 80
    sc_chunks = 25

    n_super = n_edges // NW // c_sz // sc_chunks
    sidx3 = sidx.astype(jnp.int32).reshape(NW, n_super, sc_chunks, c_sz)
    tidx3 = tidx.astype(jnp.int32).reshape(NW, n_super, sc_chunks, c_sz)

    partials = _sc_scatter_gather(n_nodes, n_edges, d, c_sz, sc_chunks, 3)(
        input, sidx3, tidx3, enorm, esgn)
    return _tc_add(n_nodes, d, 2000)(partials[:n_nodes], partials[n_nodes:])


# speedup vs baseline: 1.1764x; 1.0000x over previous
import functools

import jax
import jax.numpy as jnp
from jax import lax
from jax.experimental import pallas as pl
from jax.experimental.pallas import tpu as pltpu
from jax.experimental.pallas import tpu_sc as plsc

NC = 2
NS = 16
NW = NC * NS
L = 16


def _sc_scatter_gather(n_nodes, n_edges, d, c_sz, sc_chunks, nbuf):
    epw = n_edges // NW
    n_chunks = epw // c_sz
    n_super = n_chunks // sc_chunks
    s_sz = sc_chunks * c_sz
    rows_pt = (n_nodes // NS) // 8 * 8
    tail = n_nodes - rows_pt * NS
    zrows = 8
    mesh = plsc.VectorSubcoreMesh(core_axis_name="c", subcore_axis_name="s")

    @functools.partial(
        pl.kernel,
        out_type=jax.ShapeDtypeStruct((NC * n_nodes, d), jnp.float32),
        mesh=mesh,
        scratch_types=(
            [
                pltpu.VMEM((sc_chunks, c_sz), jnp.int32),
                pltpu.VMEM((sc_chunks, c_sz), jnp.int32),
                pltpu.VMEM((s_sz,), jnp.float32),
                pltpu.VMEM((s_sz,), jnp.float32),
            ]
            + [pltpu.VMEM((c_sz, d), jnp.float32)] * nbuf
            + [
                pltpu.VMEM((zrows, d), jnp.float32),
                pltpu.VMEM_SHARED((n_nodes, d), jnp.float32),
            ]
            + [pltpu.SemaphoreType.DMA] * nbuf
            + [pltpu.SemaphoreType.DMA]
            + [pltpu.SemaphoreType.DMA]
        ),
    )
    def k(inp_hbm, sidx_hbm, tidx_hbm, enorm_hbm, esgn_hbm, out_hbm, *scr):
        sidx_v, tidx_v, w_v, sg_v = scr[:4]
        rows = scr[4:4 + nbuf]
        zbuf = scr[4 + nbuf]
        acc = scr[5 + nbuf]
        gs = scr[6 + nbuf:6 + 2 * nbuf]
        aux_sem = scr[6 + 2 * nbuf]
        stg_sem = scr[7 + 2 * nbuf]
        cid = lax.axis_index("c")
        sid = lax.axis_index("s")
        wid = sid * NC + cid
        row0 = pl.multiple_of(sid * rows_pt, 8)

        n_zcopies = rows_pt // zrows
        def zero_zbuf(r, _):
            for f in range(d // L):
                zbuf[r, pl.ds(f * L, L)] = jnp.zeros((L,), jnp.float32)
            return 0
        lax.fori_loop(0, zrows, zero_zbuf, 0)

        def zero_acc(i, _):
            pltpu.async_copy(
                zbuf, acc.at[pl.ds(pl.multiple_of(row0 + i * zrows, 8), zrows)],
                aux_sem)
            return 0
        lax.fori_loop(0, n_zcopies, zero_acc, 0)
        if tail:
            @pl.when(sid == NS - 1)
            def _():
                for tpart in range(0, tail, zrows):
                    pltpu.async_copy(
                        zbuf, acc.at[pl.ds(NS * rows_pt + tpart, zrows)],
                        aux_sem)

        def drain_zero(i, _):
            pltpu.make_async_copy(
                zbuf, acc.at[pl.ds(0, zrows)], aux_sem).wait()
            return 0

        def scale(rows_v, c):
            def escale(g, _):
                w16 = w_v[pl.ds(c * c_sz + g * L, L)]
                for j in range(L):
                    wb = lax.gather(
                        w16, jnp.full((L, 1), j, jnp.int32),
                        lax.GatherDimensionNumbers(
                            offset_dims=(), collapsed_slice_dims=(0,),
                            start_index_map=(0,)),
                        (1,), mode=lax.GatherScatterMode.PROMISE_IN_BOUNDS)
                    e = g * L + j
                    for f in range(d // L):
                        rows_v[e, pl.ds(f * L, L)] = (
                            rows_v[e, pl.ds(f * L, L)] * wb)
                return 0
            lax.fori_loop(0, c_sz // L, escale, 0)

        def issue_gather(c, rows_v, sem):
            pltpu.async_copy(inp_hbm.at[sidx_v.at[c]], rows_v, sem)

        def wait_gather(rows_v, sem):
            pltpu.make_async_copy(inp_hbm.at[sidx_v.at[0]], rows_v, sem).wait()

        def issue_scatter(c, rows_v):
            pltpu.sync_copy(rows_v, acc.at[tidx_v.at[c]], add=True)

        def super_round(s, _):
            e0 = pl.multiple_of(wid * epw + s * s_sz, 8)
            pltpu.async_copy(sidx_hbm.at[wid, s], sidx_v, stg_sem)
            pltpu.async_copy(tidx_hbm.at[wid, s], tidx_v, stg_sem)
            pltpu.async_copy(enorm_hbm.at[pl.ds(e0, s_sz)], w_v, stg_sem)
            pltpu.async_copy(esgn_hbm.at[pl.ds(e0, s_sz)], sg_v, stg_sem)
            pltpu.make_async_copy(sidx_hbm.at[wid, 0], sidx_v, stg_sem).wait()
            pltpu.make_async_copy(tidx_hbm.at[wid, 0], tidx_v, stg_sem).wait()
            pltpu.make_async_copy(
                enorm_hbm.at[pl.ds(0, s_sz)], w_v, stg_sem).wait()
            pltpu.make_async_copy(
                esgn_hbm.at[pl.ds(0, s_sz)], sg_v, stg_sem).wait()

            def wmul(kk, _):
                w_v[pl.ds(kk * L, L)] = (
                    w_v[pl.ds(kk * L, L)] * sg_v[pl.ds(kk * L, L)])
                return 0
            lax.fori_loop(0, s_sz // L, wmul, 0)

            for b in range(min(nbuf - 1, sc_chunks)):
                issue_gather(b, rows[b], gs[b])

            @pl.when(s == 0)
            def _():
                lax.fori_loop(0, n_zcopies, drain_zero, 0)
                if tail:
                    @pl.when(sid == NS - 1)
                    def _():
                        for _i in range(0, tail, zrows):
                            pltpu.make_async_copy(
                                zbuf, acc.at[pl.ds(0, zrows)], aux_sem).wait()
                plsc.subcore_barrier()

            def phase(c, b):
                @pl.when(c < sc_chunks)
                def _():
                    b2 = (b + nbuf - 1) % nbuf
                    @pl.when(c + nbuf - 1 < sc_chunks)
                    def _():
                        issue_gather(c + nbuf - 1, rows[b2], gs[b2])
                    wait_gather(rows[b], gs[b])
                    scale(rows[b], c)
                    issue_scatter(c, rows[b])

            def group(t, _):
                for b in range(nbuf):
                    phase(nbuf * t + b, b)
                return 0
            lax.fori_loop(0, (sc_chunks + nbuf - 1) // nbuf, group, 0)
            return 0
        lax.fori_loop(0, n_super, super_round, 0)

        plsc.subcore_barrier()

        pltpu.sync_copy(
            acc.at[pl.ds(row0, rows_pt)],
            out_hbm.at[pl.ds(
                pl.multiple_of(cid * n_nodes + row0, 8), rows_pt)])
        if tail:
            @pl.when(sid == NS - 1)
            def _():
                pltpu.sync_copy(
                    acc.at[pl.ds(NS * rows_pt, tail)],
                    out_hbm.at[pl.ds(pl.multiple_of(
                        cid * n_nodes + NS * rows_pt, 8), tail)])

    return k


def _tc_add(n_nodes, d, blk):
    def body(a_ref, b_ref, o_ref):
        o_ref[...] = a_ref[...] + b_ref[...]

    return pl.pallas_call(
        body,
        grid=(n_nodes // blk,),
        in_specs=[pl.BlockSpec((blk, d), lambda i: (i, 0))] * 2,
        out_specs=pl.BlockSpec((blk, d), lambda i: (i, 0)),
        out_shape=jax.ShapeDtypeStruct((n_nodes, d), jnp.float32),
    )


@jax.jit
def kernel(input, sidx, tidx, enorm, esgn):
    n_nodes, d = input.shape
    n_edges = sidx.shape[0]
    c_sz = 80
    sc_chunks = 25

    n_super = n_edges // NW // c_sz // sc_chunks
    sidx3 = sidx.astype(jnp.int32).reshape(NW, n_super, sc_chunks, c_sz)
    tidx3 = tidx.astype(jnp.int32).reshape(NW, n_super, sc_chunks, c_sz)

    partials = _sc_scatter_gather(n_nodes, n_edges, d, c_sz, sc_chunks, 3)(
        input, sidx3, tidx3, enorm, esgn)
    return _tc_add(n_nodes, d, 2000)(partials[:n_nodes], partials[n_nodes:])
